# Initial kernel scaffold; baseline (speedup 1.0000x reference)
import jax
import jax.numpy as jnp
from jax.experimental import pallas as pl


def kernel(points, features, lorentz_vectors, mask, W_emb, b_emb, W_gat, al_gat, ar_gat, b_gat, g_bn, be_bn, W_last, al_last, ar_last, b_last, g_bnl, be_bnl, W1, b1, W2, b2, W3, b3):
    raise NotImplementedError("write your pallas kernel here")


# speedup vs baseline: 24.5967x; 24.5967x over previous
import jax
import jax.numpy as jnp
from jax.experimental import pallas as pl
from jax.experimental.pallas import tpu as pltpu

_B, _N, _K = 64, 128, 16
_H, _D = 4, 70
_HID = _H * _D
_NT = 34
_NL = 9
_S = 4
_G = _B // _S
_ROWS = _S * _N
_NTOT = _B * _N

_INTERPRET = False


def _elu(x):
    return jnp.where(x > 0, x, jnp.exp(jnp.minimum(x, 0.0)) - 1.0)


def _leaky(x):
    return jnp.where(x >= 0, x, 0.2 * x)


def _gat_body(pts_ref, feat_ref, mask_ref, wemb_ref, bemb_ref, wgat_ref,
              al_ref, ar_ref, bgat_ref, gbn_ref, bebn_ref, wlast_ref,
              allast_ref, arlast_ref, blast_ref, gbnl_ref, bebnl_ref,
              w1_ref, b1_ref, w2_ref, b2_ref, w3_ref, b3_ref,
              y_ref, H_s, P_s, A_s, accs_s, accq_s, mean_s, inv_s):
    i = pl.program_id(0)
    g = pl.program_id(1)
    row0 = g * _ROWS

    @pl.when(jnp.logical_and(i == 0, g == 0))
    def _init_acc():
        accs_s[...] = jnp.zeros_like(accs_s)
        accq_s[...] = jnp.zeros_like(accq_s)

    @pl.when(jnp.logical_and(i > 0, g == 0))
    def _finalize_stats():
        m = accs_s[...] * (1.0 / _NTOT)
        q = accq_s[...] * (1.0 / _NTOT)
        mean_s[...] = m
        inv_s[...] = 1.0 / jnp.sqrt(q - m * m + 1e-5)
        accs_s[...] = jnp.zeros_like(accs_s)
        accq_s[...] = jnp.zeros_like(accq_s)

    @pl.when(i == 0)
    def _knn_and_embed():
        pts = pts_ref[...]
        px = pts[:, :, 0]
        py = pts[:, :, 1]
        dx = px[:, :, None] - px[:, None, :]
        dy = py[:, :, None] - py[:, None, :]
        d = dx * dx + dy * dy
        n_i = jax.lax.broadcasted_iota(jnp.int32, (_S, _N, _N), 1)
        j_i = jax.lax.broadcasted_iota(jnp.int32, (_S, _N, _N), 2)
        d = d + jnp.where(n_i == j_i, 1e9, 0.0)
        jf = j_i.astype(jnp.float32)

        def _extract(_, carry):
            adj_c, cur_c = carry
            mval = jnp.min(cur_c, axis=2, keepdims=True)
            cand = jnp.where(cur_c == mval, jf, 1e9)
            jstar = jnp.min(cand, axis=2, keepdims=True)
            pick = jf == jstar
            return (jnp.where(pick, 1.0, adj_c),
                    jnp.where(pick, 3e9, cur_c))

        adj, _ = jax.lax.fori_loop(
            0, _K, _extract, (jnp.zeros((_S, _N, _N), jnp.float32), d))
        A_s[pl.ds(g * _S, _S), :, :] = adj.astype(jnp.bfloat16)
        feat = feat_ref[...].reshape(_ROWS, _NT)
        h0 = jnp.dot(feat, wemb_ref[...],
                     preferred_element_type=jnp.float32) + bemb_ref[...]
        H_s[pl.ds(row0, _ROWS), :] = h0

    @pl.when(jnp.logical_and(i >= 1, i <= _NL - 1))
    def _apply_bn_residual():
        p = P_s[pl.ds(row0, _ROWS), :]
        x = gbn_ref[0] * (p - mean_s[...]) * inv_s[...] + bebn_ref[0]
        H_s[pl.ds(row0, _ROWS), :] = H_s[pl.ds(row0, _ROWS), :] + _elu(x)

    @pl.when(i <= _NL - 2)
    def _conv_multihead():
        h = H_s[pl.ds(row0, _ROWS), :]
        z = jnp.dot(h, wgat_ref[0], preferred_element_type=jnp.float32)
        z3 = z.reshape(_S, _N, _HID)
        al_row = al_ref[0].reshape(1, 1, _HID)
        ar_row = ar_ref[0].reshape(1, 1, _HID)
        adj = A_s[pl.ds(g * _S, _S), :, :].astype(jnp.float32)
        cols = [[None] * _H for _ in range(_S)]
        for hh in range(_H):
            sl = slice(hh * _D, (hh + 1) * _D)
            el = jnp.sum(z3[:, :, sl] * al_row[:, :, sl], axis=2)
            er = jnp.sum(z3[:, :, sl] * ar_row[:, :, sl], axis=2)
            e = _leaky(el[:, None, :] + er[:, :, None])
            em = jnp.where(adj > 0, e, -1e9)
            emax = jnp.max(em, axis=2, keepdims=True)
            ee = jnp.exp(em - emax) * adj
            den = jnp.sum(ee, axis=2, keepdims=True)
            alpha = ee / (den + 1e-9)
            for b in range(_S):
                cols[b][hh] = jnp.dot(alpha[b], z3[b, :, sl],
                                      preferred_element_type=jnp.float32)
        for b in range(_S):
            rowblk = jnp.concatenate(cols[b], axis=1) + bgat_ref[0]
            P_s[pl.ds(row0 + b * _N, _N), :] = rowblk
            accs_s[...] += jnp.sum(rowblk, axis=0, keepdims=True)
            accq_s[...] += jnp.sum(rowblk * rowblk, axis=0, keepdims=True)

    @pl.when(i == _NL - 1)
    def _conv_last():
        h = H_s[pl.ds(row0, _ROWS), :]
        z = jnp.dot(h, wlast_ref[...], preferred_element_type=jnp.float32)
        z3 = z.reshape(_S, _N, _D)
        el = jnp.sum(z3 * allast_ref[...].reshape(1, 1, _D), axis=2)
        er = jnp.sum(z3 * arlast_ref[...].reshape(1, 1, _D), axis=2)
        adj = A_s[pl.ds(g * _S, _S), :, :].astype(jnp.float32)
        e = _leaky(el[:, None, :] + er[:, :, None])
        em = jnp.where(adj > 0, e, -1e9)
        emax = jnp.max(em, axis=2, keepdims=True)
        ee = jnp.exp(em - emax) * adj
        den = jnp.sum(ee, axis=2, keepdims=True)
        alpha = ee / (den + 1e-9)
        zero_pad = jnp.zeros((_N, _HID - _D), jnp.float32)
        for b in range(_S):
            ob = jnp.dot(alpha[b], z3[b],
                         preferred_element_type=jnp.float32) + blast_ref[...]
            rowblk = jnp.concatenate([ob, zero_pad], axis=1)
            P_s[pl.ds(row0 + b * _N, _N), :] = rowblk
            accs_s[...] += jnp.sum(rowblk, axis=0, keepdims=True)
            accq_s[...] += jnp.sum(rowblk * rowblk, axis=0, keepdims=True)

    @pl.when(i == _NL)
    def _readout():
        p = P_s[pl.ds(row0, _ROWS), :][:, 0:_D]
        x = (gbnl_ref[...] * (p - mean_s[0:1, 0:_D]) * inv_s[0:1, 0:_D]
             + bebnl_ref[...])
        hn = _elu(x).reshape(_S, _N, _D)
        mk = mask_ref[...].reshape(_S, _N)
        tot = jnp.sum(hn * mk[:, :, None], axis=1)
        cnt = jnp.sum(mk, axis=1, keepdims=True) + 1e-9
        hg = tot / cnt
        y1 = jnp.maximum(jnp.dot(hg, w1_ref[...],
                                 preferred_element_type=jnp.float32)
                         + b1_ref[...], 0.0)
        y2 = jnp.maximum(jnp.dot(y1, w2_ref[...],
                                 preferred_element_type=jnp.float32)
                         + b2_ref[...], 0.0)
        y_ref[pl.ds(g * _S, _S), :] = jnp.dot(
            y2, w3_ref[...], preferred_element_type=jnp.float32) + b3_ref[...]


def kernel(points, features, lorentz_vectors, mask, W_emb, b_emb, W_gat,
           al_gat, ar_gat, b_gat, g_bn, be_bn, W_last, al_last, ar_last,
           b_last, g_bnl, be_bnl, W1, b1, W2, b2, W3, b3):
    del lorentz_vectors
    f32 = jnp.float32
    row = lambda v: v.reshape(1, -1)
    lay = lambda v: v.reshape(_NL - 1, 1, -1)

    full = lambda shp: pl.BlockSpec(shp, lambda i, g: tuple(0 for _ in shp))
    grp = lambda shp: pl.BlockSpec(
        shp, lambda i, g: (g,) + tuple(0 for _ in shp[1:]))
    per_layer = lambda shp: pl.BlockSpec(
        shp, lambda i, g: (jnp.minimum(i, _NL - 2),) + tuple(0 for _ in shp[1:]))
    prev_layer = lambda shp: pl.BlockSpec(
        shp, lambda i, g: (jnp.clip(i - 1, 0, _NL - 2),)
        + tuple(0 for _ in shp[1:]))

    out = pl.pallas_call(
        _gat_body,
        grid=(_NL + 1, _G),
        in_specs=[
            grp((_S, _N, 2)),
            grp((_S, _N, _NT)),
            grp((_S, 1, _N)),
            full((_NT, _HID)),
            full((1, _HID)),
            per_layer((1, _HID, _HID)),
            per_layer((1, 1, _HID)),
            per_layer((1, 1, _HID)),
            per_layer((1, 1, _HID)),
            prev_layer((1, 1, _HID)),
            prev_layer((1, 1, _HID)),
            full((_HID, _D)),
            full((1, _D)),
            full((1, _D)),
            full((1, _D)),
            full((1, _D)),
            full((1, _D)),
            full((_D, 35)),
            full((1, 35)),
            full((35, 17)),
            full((1, 17)),
            full((17, 5)),
            full((1, 5)),
        ],
        out_specs=pl.BlockSpec((_B, 5), lambda i, g: (0, 0)),
        out_shape=jax.ShapeDtypeStruct((_B, 5), f32),
        scratch_shapes=[
            pltpu.VMEM((_NTOT, _HID), f32),
            pltpu.VMEM((_NTOT, _HID), f32),
            pltpu.VMEM((_B, _N, _N), jnp.bfloat16),
            pltpu.VMEM((1, _HID), f32),
            pltpu.VMEM((1, _HID), f32),
            pltpu.VMEM((1, _HID), f32),
            pltpu.VMEM((1, _HID), f32),
        ],
        interpret=_INTERPRET,
    )(points.astype(f32), features.astype(f32),
      mask.astype(f32).reshape(_B, 1, _N),
      W_emb, row(b_emb), W_gat, lay(al_gat), lay(ar_gat), lay(b_gat),
      lay(g_bn), lay(be_bn), W_last, al_last, ar_last, row(b_last),
      row(g_bnl), row(be_bnl), W1, row(b1), W2, row(b2), W3, row(b3))
    return out


# speedup vs baseline: 235.6782x; 9.5817x over previous
import jax
import jax.numpy as jnp
from jax.experimental import pallas as pl
from jax.experimental.pallas import tpu as pltpu

_B, _N, _K = 64, 128, 16
_H, _D = 4, 70
_HID = _H * _D
_NT = 34
_NL = 9
_S = 4
_G = _B // _S
_ROWS = _S * _N
_NTOT = _B * _N

_INTERPRET = False


def _elu(x):
    return jnp.where(x > 0, x, jnp.exp(jnp.minimum(x, 0.0)) - 1.0)


def _leaky(x):
    return jnp.where(x >= 0, x, 0.2 * x)


def _eye_n():
    r = jax.lax.broadcasted_iota(jnp.int32, (_N, _N), 0)
    c = jax.lax.broadcasted_iota(jnp.int32, (_N, _N), 1)
    return jnp.where(r == c, 1.0, 0.0).astype(jnp.float32)


def _transpose_cols(cols, ident):
    return jax.lax.dot_general(cols, ident, (((0,), (0,)), ((), ())),
                               preferred_element_type=jnp.float32,
                               precision=jax.lax.Precision.HIGHEST)


def _gat_body(pts_ref, feat_ref, mask_ref, wemb_ref, bemb_ref, wgat_ref,
              bgat_ref, gbn_ref, bebn_ref, wlast_ref,
              blast_ref, gbnl_ref, bebnl_ref,
              w1_ref, b1_ref, w2_ref, b2_ref, w3_ref, b3_ref,
              y_ref, H_s, P_s, A_s, accs_s, accq_s, mean_s, inv_s):
    i = pl.program_id(0)
    g = pl.program_id(1)
    row0 = g * _ROWS

    @pl.when(jnp.logical_and(i == 0, g == 0))
    def _init_acc():
        accs_s[...] = jnp.zeros_like(accs_s)
        accq_s[...] = jnp.zeros_like(accq_s)

    @pl.when(jnp.logical_and(i > 0, g == 0))
    def _finalize_stats():
        m = accs_s[...] * (1.0 / _NTOT)
        q = accq_s[...] * (1.0 / _NTOT)
        mean_s[...] = m
        inv_s[...] = 1.0 / jnp.sqrt(q - m * m + 1e-5)
        accs_s[...] = jnp.zeros_like(accs_s)
        accq_s[...] = jnp.zeros_like(accq_s)

    @pl.when(i == 0)
    def _knn_and_embed():
        pts = pts_ref[...]
        ident = _eye_n()
        jf = jax.lax.broadcasted_iota(jnp.int32, (_N, _N), 1).astype(
            jnp.float32)
        eye_big = jnp.where(
            jax.lax.broadcasted_iota(jnp.int32, (_N, _N), 0)
            == jax.lax.broadcasted_iota(jnp.int32, (_N, _N), 1), 1e9, 0.0)

        def _extract(_, carry):
            adj_c, cur_c = carry
            mval = jnp.min(cur_c, axis=1, keepdims=True)
            cand = jnp.where(cur_c == mval, jf, 1e9)
            jstar = jnp.min(cand, axis=1, keepdims=True)
            pick = jf == jstar
            return (jnp.where(pick, 1.0, adj_c),
                    jnp.where(pick, 3e9, cur_c))

        for b in range(_S):
            pc = pts[b]
            pr = _transpose_cols(pc, ident)
            dx = pc[:, 0:1] - pr[0:1, :]
            dy = pc[:, 1:2] - pr[1:2, :]
            d = dx * dx + dy * dy + eye_big
            adj, _ = jax.lax.fori_loop(
                0, _K, _extract, (jnp.zeros((_N, _N), jnp.float32), d))
            A_s[pl.ds(g * _S + b, 1), :, :] = adj.reshape(
                1, _N, _N).astype(jnp.bfloat16)
        feat = feat_ref[...].reshape(_ROWS, _NT)
        h0 = jnp.dot(feat, wemb_ref[...],
                     preferred_element_type=jnp.float32) + bemb_ref[...]
        H_s[pl.ds(row0, _ROWS), :] = h0

    @pl.when(jnp.logical_and(i >= 1, i <= _NL - 1))
    def _apply_bn_residual():
        p = P_s[pl.ds(row0, _ROWS), :]
        x = gbn_ref[0] * (p - mean_s[...]) * inv_s[...] + bebn_ref[0]
        H_s[pl.ds(row0, _ROWS), :] = H_s[pl.ds(row0, _ROWS), :] + _elu(x)

    @pl.when(i <= _NL - 2)
    def _conv_multihead():
        h = H_s[pl.ds(row0, _ROWS), :]
        zext = jnp.dot(h, wgat_ref[0], preferred_element_type=jnp.float32)
        z3e = zext.reshape(_S, _N, _HID + 2 * _H)
        ident = _eye_n()
        for b in range(_S):
            zs = z3e[b]
            adj = A_s[pl.ds(g * _S + b, 1), :, :].reshape(_N, _N)
            el_rows = _transpose_cols(zs[:, _HID:_HID + _H], ident)
            cols = []
            for hh in range(_H):
                e = _leaky(el_rows[hh:hh + 1, :]
                           + zs[:, _HID + _H + hh:_HID + _H + hh + 1])
                em = jnp.where(adj > 0, e, -1e9)
                emax = jnp.max(em, axis=1, keepdims=True)
                ee = jnp.exp(em - emax)
                den = jnp.sum(ee, axis=1, keepdims=True)
                alpha = ee / (den + 1e-9)
                cols.append(jnp.dot(alpha, zs[:, hh * _D:(hh + 1) * _D],
                                    preferred_element_type=jnp.float32))
            rowblk = jnp.concatenate(cols, axis=1) + bgat_ref[0]
            P_s[pl.ds(row0 + b * _N, _N), :] = rowblk
            accs_s[...] += jnp.sum(rowblk, axis=0, keepdims=True)
            accq_s[...] += jnp.sum(rowblk * rowblk, axis=0, keepdims=True)

    @pl.when(i == _NL - 1)
    def _conv_last():
        h = H_s[pl.ds(row0, _ROWS), :]
        zext = jnp.dot(h, wlast_ref[...], preferred_element_type=jnp.float32)
        z3e = zext.reshape(_S, _N, _D + 2)
        ident = _eye_n()
        zero_pad = jnp.zeros((_N, _HID - _D), jnp.float32)
        for b in range(_S):
            zs = z3e[b]
            adj = A_s[pl.ds(g * _S + b, 1), :, :].reshape(_N, _N)
            el_row = _transpose_cols(zs[:, _D:_D + 1], ident)
            e = _leaky(el_row + zs[:, _D + 1:_D + 2])
            em = jnp.where(adj > 0, e, -1e9)
            emax = jnp.max(em, axis=1, keepdims=True)
            ee = jnp.exp(em - emax)
            den = jnp.sum(ee, axis=1, keepdims=True)
            alpha = ee / (den + 1e-9)
            ob = jnp.dot(alpha, zs[:, 0:_D],
                         preferred_element_type=jnp.float32) + blast_ref[...]
            rowblk = jnp.concatenate([ob, zero_pad], axis=1)
            P_s[pl.ds(row0 + b * _N, _N), :] = rowblk
            accs_s[...] += jnp.sum(rowblk, axis=0, keepdims=True)
            accq_s[...] += jnp.sum(rowblk * rowblk, axis=0, keepdims=True)

    @pl.when(i == _NL)
    def _readout():
        p = P_s[pl.ds(row0, _ROWS), :][:, 0:_D]
        x = (gbnl_ref[...] * (p - mean_s[0:1, 0:_D]) * inv_s[0:1, 0:_D]
             + bebnl_ref[...])
        hn3 = _elu(x).reshape(_S, _N, _D)
        mk = mask_ref[...]
        rows = []
        for b in range(_S):
            mk_s = mk[b]
            tot = jnp.dot(mk_s, hn3[b], preferred_element_type=jnp.float32)
            cnt = jnp.sum(mk_s, axis=1, keepdims=True) + 1e-9
            rows.append(tot / cnt)
        hg = jnp.concatenate(rows, axis=0)
        y1 = jnp.maximum(jnp.dot(hg, w1_ref[...],
                                 preferred_element_type=jnp.float32)
                         + b1_ref[...], 0.0)
        y2 = jnp.maximum(jnp.dot(y1, w2_ref[...],
                                 preferred_element_type=jnp.float32)
                         + b2_ref[...], 0.0)
        y_ref[pl.ds(g * _S, _S), :] = jnp.dot(
            y2, w3_ref[...], preferred_element_type=jnp.float32) + b3_ref[...]


def kernel(points, features, lorentz_vectors, mask, W_emb, b_emb, W_gat,
           al_gat, ar_gat, b_gat, g_bn, be_bn, W_last, al_last, ar_last,
           b_last, g_bnl, be_bnl, W1, b1, W2, b2, W3, b3):
    del lorentz_vectors
    f32 = jnp.float32
    row = lambda v: v.reshape(1, -1)
    lay = lambda v: v.reshape(_NL - 1, 1, -1)

    eye_h = jnp.eye(_H, dtype=f32)
    Al = (al_gat[:, :, :, None] * eye_h[None, :, None, :]).reshape(
        _NL - 1, _HID, _H)
    Ar = (ar_gat[:, :, :, None] * eye_h[None, :, None, :]).reshape(
        _NL - 1, _HID, _H)
    W_ext = jnp.concatenate(
        [W_gat,
         jnp.einsum('lij,ljk->lik', W_gat, Al),
         jnp.einsum('lij,ljk->lik', W_gat, Ar)], axis=2)
    W_last_ext = jnp.concatenate(
        [W_last, W_last @ al_last.T, W_last @ ar_last.T], axis=1)

    full = lambda shp: pl.BlockSpec(shp, lambda i, g: tuple(0 for _ in shp))
    grp = lambda shp: pl.BlockSpec(
        shp, lambda i, g: (g,) + tuple(0 for _ in shp[1:]))
    per_layer = lambda shp: pl.BlockSpec(
        shp, lambda i, g: (jnp.minimum(i, _NL - 2),) + tuple(0 for _ in shp[1:]))
    prev_layer = lambda shp: pl.BlockSpec(
        shp, lambda i, g: (jnp.clip(i - 1, 0, _NL - 2),)
        + tuple(0 for _ in shp[1:]))

    out = pl.pallas_call(
        _gat_body,
        grid=(_NL + 1, _G),
        in_specs=[
            grp((_S, _N, 2)),
            grp((_S, _N, _NT)),
            grp((_S, 1, _N)),
            full((_NT, _HID)),
            full((1, _HID)),
            per_layer((1, _HID, _HID + 2 * _H)),
            per_layer((1, 1, _HID)),
            prev_layer((1, 1, _HID)),
            prev_layer((1, 1, _HID)),
            full((_HID, _D + 2)),
            full((1, _D)),
            full((1, _D)),
            full((1, _D)),
            full((_D, 35)),
            full((1, 35)),
            full((35, 17)),
            full((1, 17)),
            full((17, 5)),
            full((1, 5)),
        ],
        out_specs=pl.BlockSpec((_B, 5), lambda i, g: (0, 0)),
        out_shape=jax.ShapeDtypeStruct((_B, 5), f32),
        scratch_shapes=[
            pltpu.VMEM((_NTOT, _HID), f32),
            pltpu.VMEM((_NTOT, _HID), f32),
            pltpu.VMEM((_B, _N, _N), jnp.bfloat16),
            pltpu.VMEM((1, _HID), f32),
            pltpu.VMEM((1, _HID), f32),
            pltpu.VMEM((1, _HID), f32),
            pltpu.VMEM((1, _HID), f32),
        ],
        interpret=_INTERPRET,
    )(points.astype(f32), features.astype(f32),
      mask.astype(f32).reshape(_B, 1, _N),
      W_emb, row(b_emb), W_ext, lay(b_gat),
      lay(g_bn), lay(be_bn), W_last_ext, row(b_last),
      row(g_bnl), row(be_bnl), W1, row(b1), W2, row(b2), W3, row(b3))
    return out


# speedup vs baseline: 268.1561x; 1.1378x over previous
import jax
import jax.numpy as jnp
from jax.experimental import pallas as pl
from jax.experimental.pallas import tpu as pltpu

_B, _N, _K = 64, 128, 16
_H, _D = 4, 70
_HID = _H * _D
_NT = 34
_NL = 9
_S = 8
_G = _B // _S
_ROWS = _S * _N
_NTOT = _B * _N

_INTERPRET = False


def _elu(x):
    return jnp.where(x > 0, x, jnp.exp(jnp.minimum(x, 0.0)) - 1.0)


def _leaky(x):
    return jnp.where(x >= 0, x, 0.2 * x)


def _eye_n():
    r = jax.lax.broadcasted_iota(jnp.int32, (_N, _N), 0)
    c = jax.lax.broadcasted_iota(jnp.int32, (_N, _N), 1)
    return jnp.where(r == c, 1.0, 0.0).astype(jnp.float32)


def _transpose_cols(cols, ident):
    return jax.lax.dot_general(cols, ident, (((0,), (0,)), ((), ())),
                               preferred_element_type=jnp.float32,
                               precision=jax.lax.Precision.HIGHEST)


def _gat_body(pts_ref, feat_ref, mask_ref, wemb_ref, bemb_ref, wgat_ref,
              bgat_ref, gbn_ref, bebn_ref, wlast_ref,
              blast_ref, gbnl_ref, bebnl_ref,
              w1_ref, b1_ref, w2_ref, b2_ref, w3_ref, b3_ref,
              y_ref, H_s, P_s, A_s, accs_s, accq_s, mean_s, inv_s):
    i = pl.program_id(0)
    g = pl.program_id(1)
    row0 = g * _ROWS

    @pl.when(jnp.logical_and(i == 0, g == 0))
    def _init_acc():
        accs_s[...] = jnp.zeros_like(accs_s)
        accq_s[...] = jnp.zeros_like(accq_s)

    @pl.when(jnp.logical_and(i > 0, g == 0))
    def _finalize_stats():
        m = accs_s[...] * (1.0 / _NTOT)
        q = accq_s[...] * (1.0 / _NTOT)
        mean_s[...] = m
        inv_s[...] = 1.0 / jnp.sqrt(q - m * m + 1e-5)
        accs_s[...] = jnp.zeros_like(accs_s)
        accq_s[...] = jnp.zeros_like(accq_s)

    @pl.when(i == 0)
    def _knn_and_embed():
        pts = pts_ref[...]
        ident = _eye_n()
        jf = jax.lax.broadcasted_iota(jnp.int32, (_N, _N), 1).astype(
            jnp.float32)
        eye_big = jnp.where(
            jax.lax.broadcasted_iota(jnp.int32, (_N, _N), 0)
            == jax.lax.broadcasted_iota(jnp.int32, (_N, _N), 1), 1e9, 0.0)

        def _extract(_, carry):
            adj_c, cur_c = carry
            mval = jnp.min(cur_c, axis=1, keepdims=True)
            cand = jnp.where(cur_c == mval, jf, 1e9)
            jstar = jnp.min(cand, axis=1, keepdims=True)
            pick = jf == jstar
            return (jnp.where(pick, 1.0, adj_c),
                    jnp.where(pick, 3e9, cur_c))

        for b in range(_S):
            pc = pts[b]
            pr = _transpose_cols(pc, ident)
            dx = pc[:, 0:1] - pr[0:1, :]
            dy = pc[:, 1:2] - pr[1:2, :]
            d = dx * dx + dy * dy + eye_big
            adj, _ = jax.lax.fori_loop(
                0, _K, _extract, (jnp.zeros((_N, _N), jnp.float32), d))
            A_s[pl.ds(g * _S + b, 1), :, :] = adj.reshape(
                1, _N, _N).astype(jnp.bfloat16)
        feat = feat_ref[...].reshape(_ROWS, _NT)
        h0 = jnp.dot(feat, wemb_ref[...],
                     preferred_element_type=jnp.float32) + bemb_ref[...]
        H_s[pl.ds(row0, _ROWS), :] = h0

    @pl.when(jnp.logical_and(i >= 1, i <= _NL - 1))
    def _apply_bn_residual():
        p = P_s[pl.ds(row0, _ROWS), :]
        x = gbn_ref[0] * (p - mean_s[...]) * inv_s[...] + bebn_ref[0]
        H_s[pl.ds(row0, _ROWS), :] = H_s[pl.ds(row0, _ROWS), :] + _elu(x)

    @pl.when(i <= _NL - 2)
    def _conv_multihead():
        h = H_s[pl.ds(row0, _ROWS), :]
        zext = jnp.dot(h, wgat_ref[0], preferred_element_type=jnp.float32)
        z3e = zext.reshape(_S, _N, _HID + 2 * _H)
        ident = _eye_n()
        for b in range(_S):
            zs = z3e[b]
            adj = A_s[pl.ds(g * _S + b, 1), :, :].reshape(_N, _N)
            el_rows = _transpose_cols(zs[:, _HID:_HID + _H], ident)
            cols = []
            for hh in range(_H):
                e = _leaky(el_rows[hh:hh + 1, :]
                           + zs[:, _HID + _H + hh:_HID + _H + hh + 1])
                em = jnp.where(adj > 0, e, -1e9)
                emax = jnp.max(em, axis=1, keepdims=True)
                ee = jnp.exp(em - emax)
                den = jnp.sum(ee, axis=1, keepdims=True)
                alpha = ee / (den + 1e-9)
                cols.append(jnp.dot(alpha, zs[:, hh * _D:(hh + 1) * _D],
                                    preferred_element_type=jnp.float32))
            rowblk = jnp.concatenate(cols, axis=1) + bgat_ref[0]
            P_s[pl.ds(row0 + b * _N, _N), :] = rowblk
            accs_s[...] += jnp.sum(rowblk, axis=0, keepdims=True)
            accq_s[...] += jnp.sum(rowblk * rowblk, axis=0, keepdims=True)

    @pl.when(i == _NL - 1)
    def _conv_last():
        h = H_s[pl.ds(row0, _ROWS), :]
        zext = jnp.dot(h, wlast_ref[...], preferred_element_type=jnp.float32)
        z3e = zext.reshape(_S, _N, _D + 2)
        ident = _eye_n()
        zero_pad = jnp.zeros((_N, _HID - _D), jnp.float32)
        for b in range(_S):
            zs = z3e[b]
            adj = A_s[pl.ds(g * _S + b, 1), :, :].reshape(_N, _N)
            el_row = _transpose_cols(zs[:, _D:_D + 1], ident)
            e = _leaky(el_row + zs[:, _D + 1:_D + 2])
            em = jnp.where(adj > 0, e, -1e9)
            emax = jnp.max(em, axis=1, keepdims=True)
            ee = jnp.exp(em - emax)
            den = jnp.sum(ee, axis=1, keepdims=True)
            alpha = ee / (den + 1e-9)
            ob = jnp.dot(alpha, zs[:, 0:_D],
                         preferred_element_type=jnp.float32) + blast_ref[...]
            rowblk = jnp.concatenate([ob, zero_pad], axis=1)
            P_s[pl.ds(row0 + b * _N, _N), :] = rowblk
            accs_s[...] += jnp.sum(rowblk, axis=0, keepdims=True)
            accq_s[...] += jnp.sum(rowblk * rowblk, axis=0, keepdims=True)

    @pl.when(i == _NL)
    def _readout():
        p = P_s[pl.ds(row0, _ROWS), :][:, 0:_D]
        x = (gbnl_ref[...] * (p - mean_s[0:1, 0:_D]) * inv_s[0:1, 0:_D]
             + bebnl_ref[...])
        hn3 = _elu(x).reshape(_S, _N, _D)
        mk = mask_ref[...]
        rows = []
        for b in range(_S):
            mk_s = mk[b]
            tot = jnp.dot(mk_s, hn3[b], preferred_element_type=jnp.float32)
            cnt = jnp.sum(mk_s, axis=1, keepdims=True) + 1e-9
            rows.append(tot / cnt)
        hg = jnp.concatenate(rows, axis=0)
        y1 = jnp.maximum(jnp.dot(hg, w1_ref[...],
                                 preferred_element_type=jnp.float32)
                         + b1_ref[...], 0.0)
        y2 = jnp.maximum(jnp.dot(y1, w2_ref[...],
                                 preferred_element_type=jnp.float32)
                         + b2_ref[...], 0.0)
        y_ref[pl.ds(g * _S, _S), :] = jnp.dot(
            y2, w3_ref[...], preferred_element_type=jnp.float32) + b3_ref[...]


def kernel(points, features, lorentz_vectors, mask, W_emb, b_emb, W_gat,
           al_gat, ar_gat, b_gat, g_bn, be_bn, W_last, al_last, ar_last,
           b_last, g_bnl, be_bnl, W1, b1, W2, b2, W3, b3):
    del lorentz_vectors
    f32 = jnp.float32
    row = lambda v: v.reshape(1, -1)
    lay = lambda v: v.reshape(_NL - 1, 1, -1)

    eye_h = jnp.eye(_H, dtype=f32)
    Al = (al_gat[:, :, :, None] * eye_h[None, :, None, :]).reshape(
        _NL - 1, _HID, _H)
    Ar = (ar_gat[:, :, :, None] * eye_h[None, :, None, :]).reshape(
        _NL - 1, _HID, _H)
    W_ext = jnp.concatenate(
        [W_gat,
         jnp.einsum('lij,ljk->lik', W_gat, Al),
         jnp.einsum('lij,ljk->lik', W_gat, Ar)], axis=2)
    W_last_ext = jnp.concatenate(
        [W_last, W_last @ al_last.T, W_last @ ar_last.T], axis=1)

    full = lambda shp: pl.BlockSpec(shp, lambda i, g: tuple(0 for _ in shp))
    grp = lambda shp: pl.BlockSpec(
        shp, lambda i, g: (g,) + tuple(0 for _ in shp[1:]))
    per_layer = lambda shp: pl.BlockSpec(
        shp, lambda i, g: (jnp.minimum(i, _NL - 2),) + tuple(0 for _ in shp[1:]))
    prev_layer = lambda shp: pl.BlockSpec(
        shp, lambda i, g: (jnp.clip(i - 1, 0, _NL - 2),)
        + tuple(0 for _ in shp[1:]))

    out = pl.pallas_call(
        _gat_body,
        grid=(_NL + 1, _G),
        in_specs=[
            grp((_S, _N, 2)),
            grp((_S, _N, _NT)),
            grp((_S, 1, _N)),
            full((_NT, _HID)),
            full((1, _HID)),
            per_layer((1, _HID, _HID + 2 * _H)),
            per_layer((1, 1, _HID)),
            prev_layer((1, 1, _HID)),
            prev_layer((1, 1, _HID)),
            full((_HID, _D + 2)),
            full((1, _D)),
            full((1, _D)),
            full((1, _D)),
            full((_D, 35)),
            full((1, 35)),
            full((35, 17)),
            full((1, 17)),
            full((17, 5)),
            full((1, 5)),
        ],
        out_specs=pl.BlockSpec((_B, 5), lambda i, g: (0, 0)),
        out_shape=jax.ShapeDtypeStruct((_B, 5), f32),
        scratch_shapes=[
            pltpu.VMEM((_NTOT, _HID), f32),
            pltpu.VMEM((_NTOT, _HID), f32),
            pltpu.VMEM((_B, _N, _N), jnp.bfloat16),
            pltpu.VMEM((1, _HID), f32),
            pltpu.VMEM((1, _HID), f32),
            pltpu.VMEM((1, _HID), f32),
            pltpu.VMEM((1, _HID), f32),
        ],
        interpret=_INTERPRET,
    )(points.astype(f32), features.astype(f32),
      mask.astype(f32).reshape(_B, 1, _N),
      W_emb, row(b_emb), W_ext, lay(b_gat),
      lay(g_bn), lay(be_bn), W_last_ext, row(b_last),
      row(g_bnl), row(be_bnl), W1, row(b1), W2, row(b2), W3, row(b3))
    return out


# speedup vs baseline: 283.3024x; 1.0565x over previous
import jax
import jax.numpy as jnp
from jax.experimental import pallas as pl
from jax.experimental.pallas import tpu as pltpu

_B, _N, _K = 64, 128, 16
_H, _D = 4, 70
_HID = _H * _D
_NT = 34
_NL = 9
_S = 16
_G = _B // _S
_ROWS = _S * _N
_NTOT = _B * _N

_INTERPRET = False


def _elu(x):
    return jnp.where(x > 0, x, jnp.exp(jnp.minimum(x, 0.0)) - 1.0)


def _leaky(x):
    return jnp.where(x >= 0, x, 0.2 * x)


def _eye_n():
    r = jax.lax.broadcasted_iota(jnp.int32, (_N, _N), 0)
    c = jax.lax.broadcasted_iota(jnp.int32, (_N, _N), 1)
    return jnp.where(r == c, 1.0, 0.0).astype(jnp.float32)


def _transpose_cols(cols, ident):
    return jax.lax.dot_general(cols, ident, (((0,), (0,)), ((), ())),
                               preferred_element_type=jnp.float32,
                               precision=jax.lax.Precision.HIGHEST)


def _gat_body(pts_ref, feat_ref, mask_ref, wemb_ref, bemb_ref, wgat_ref,
              bgat_ref, gbn_ref, bebn_ref, wlast_ref,
              blast_ref, gbnl_ref, bebnl_ref,
              w1_ref, b1_ref, w2_ref, b2_ref, w3_ref, b3_ref,
              y_ref, H_s, P_s, A_s, accs_s, accq_s, mean_s, inv_s):
    i = pl.program_id(0)
    g = pl.program_id(1)
    row0 = g * _ROWS

    @pl.when(jnp.logical_and(i == 0, g == 0))
    def _init_acc():
        accs_s[...] = jnp.zeros_like(accs_s)
        accq_s[...] = jnp.zeros_like(accq_s)

    @pl.when(jnp.logical_and(i > 0, g == 0))
    def _finalize_stats():
        m = accs_s[...] * (1.0 / _NTOT)
        q = accq_s[...] * (1.0 / _NTOT)
        mean_s[...] = m
        inv_s[...] = 1.0 / jnp.sqrt(q - m * m + 1e-5)
        accs_s[...] = jnp.zeros_like(accs_s)
        accq_s[...] = jnp.zeros_like(accq_s)

    @pl.when(i == 0)
    def _knn_and_embed():
        pts = pts_ref[...]
        ident = _eye_n()
        jf = jax.lax.broadcasted_iota(jnp.int32, (_N, _N), 1).astype(
            jnp.float32)
        eye_big = jnp.where(
            jax.lax.broadcasted_iota(jnp.int32, (_N, _N), 0)
            == jax.lax.broadcasted_iota(jnp.int32, (_N, _N), 1), 1e9, 0.0)

        def _extract(_, carry):
            adj_c, cur_c = carry
            mval = jnp.min(cur_c, axis=1, keepdims=True)
            cand = jnp.where(cur_c == mval, jf, 1e9)
            jstar = jnp.min(cand, axis=1, keepdims=True)
            pick = jf == jstar
            return (jnp.where(pick, 1.0, adj_c),
                    jnp.where(pick, 3e9, cur_c))

        for b in range(_S):
            pc = pts[b]
            pr = _transpose_cols(pc, ident)
            dx = pc[:, 0:1] - pr[0:1, :]
            dy = pc[:, 1:2] - pr[1:2, :]
            d = dx * dx + dy * dy + eye_big
            adj, _ = jax.lax.fori_loop(
                0, _K, _extract, (jnp.zeros((_N, _N), jnp.float32), d))
            A_s[pl.ds(g * _S + b, 1), :, :] = adj.reshape(
                1, _N, _N).astype(jnp.bfloat16)
        feat = feat_ref[...].reshape(_ROWS, _NT)
        h0 = jnp.dot(feat, wemb_ref[...],
                     preferred_element_type=jnp.float32) + bemb_ref[...]
        H_s[pl.ds(row0, _ROWS), :] = h0

    @pl.when(jnp.logical_and(i >= 1, i <= _NL - 1))
    def _apply_bn_residual():
        p = P_s[pl.ds(row0, _ROWS), :]
        x = gbn_ref[0] * (p - mean_s[...]) * inv_s[...] + bebn_ref[0]
        H_s[pl.ds(row0, _ROWS), :] = H_s[pl.ds(row0, _ROWS), :] + _elu(x)

    @pl.when(i <= _NL - 2)
    def _conv_multihead():
        h = H_s[pl.ds(row0, _ROWS), :]
        zext = jnp.dot(h, wgat_ref[0], preferred_element_type=jnp.float32)
        z3e = zext.reshape(_S, _N, _HID + 2 * _H)
        ident = _eye_n()
        for b in range(_S):
            zs = z3e[b]
            adj = A_s[pl.ds(g * _S + b, 1), :, :].reshape(_N, _N)
            el_rows = _transpose_cols(zs[:, _HID:_HID + _H], ident)
            cols = []
            for hh in range(_H):
                e = _leaky(el_rows[hh:hh + 1, :]
                           + zs[:, _HID + _H + hh:_HID + _H + hh + 1])
                em = jnp.where(adj > 0, e, -1e9)
                emax = jnp.max(em, axis=1, keepdims=True)
                ee = jnp.exp(em - emax)
                den = jnp.sum(ee, axis=1, keepdims=True)
                alpha = ee / (den + 1e-9)
                cols.append(jnp.dot(alpha, zs[:, hh * _D:(hh + 1) * _D],
                                    preferred_element_type=jnp.float32))
            rowblk = jnp.concatenate(cols, axis=1) + bgat_ref[0]
            P_s[pl.ds(row0 + b * _N, _N), :] = rowblk
            accs_s[...] += jnp.sum(rowblk, axis=0, keepdims=True)
            accq_s[...] += jnp.sum(rowblk * rowblk, axis=0, keepdims=True)

    @pl.when(i == _NL - 1)
    def _conv_last():
        h = H_s[pl.ds(row0, _ROWS), :]
        zext = jnp.dot(h, wlast_ref[...], preferred_element_type=jnp.float32)
        z3e = zext.reshape(_S, _N, _D + 2)
        ident = _eye_n()
        zero_pad = jnp.zeros((_N, _HID - _D), jnp.float32)
        for b in range(_S):
            zs = z3e[b]
            adj = A_s[pl.ds(g * _S + b, 1), :, :].reshape(_N, _N)
            el_row = _transpose_cols(zs[:, _D:_D + 1], ident)
            e = _leaky(el_row + zs[:, _D + 1:_D + 2])
            em = jnp.where(adj > 0, e, -1e9)
            emax = jnp.max(em, axis=1, keepdims=True)
            ee = jnp.exp(em - emax)
            den = jnp.sum(ee, axis=1, keepdims=True)
            alpha = ee / (den + 1e-9)
            ob = jnp.dot(alpha, zs[:, 0:_D],
                         preferred_element_type=jnp.float32) + blast_ref[...]
            rowblk = jnp.concatenate([ob, zero_pad], axis=1)
            P_s[pl.ds(row0 + b * _N, _N), :] = rowblk
            accs_s[...] += jnp.sum(rowblk, axis=0, keepdims=True)
            accq_s[...] += jnp.sum(rowblk * rowblk, axis=0, keepdims=True)

    @pl.when(i == _NL)
    def _readout():
        p = P_s[pl.ds(row0, _ROWS), :][:, 0:_D]
        x = (gbnl_ref[...] * (p - mean_s[0:1, 0:_D]) * inv_s[0:1, 0:_D]
             + bebnl_ref[...])
        hn3 = _elu(x).reshape(_S, _N, _D)
        mk = mask_ref[...]
        rows = []
        for b in range(_S):
            mk_s = mk[b]
            tot = jnp.dot(mk_s, hn3[b], preferred_element_type=jnp.float32)
            cnt = jnp.sum(mk_s, axis=1, keepdims=True) + 1e-9
            rows.append(tot / cnt)
        hg = jnp.concatenate(rows, axis=0)
        y1 = jnp.maximum(jnp.dot(hg, w1_ref[...],
                                 preferred_element_type=jnp.float32)
                         + b1_ref[...], 0.0)
        y2 = jnp.maximum(jnp.dot(y1, w2_ref[...],
                                 preferred_element_type=jnp.float32)
                         + b2_ref[...], 0.0)
        y_ref[pl.ds(g * _S, _S), :] = jnp.dot(
            y2, w3_ref[...], preferred_element_type=jnp.float32) + b3_ref[...]


def kernel(points, features, lorentz_vectors, mask, W_emb, b_emb, W_gat,
           al_gat, ar_gat, b_gat, g_bn, be_bn, W_last, al_last, ar_last,
           b_last, g_bnl, be_bnl, W1, b1, W2, b2, W3, b3):
    del lorentz_vectors
    f32 = jnp.float32
    row = lambda v: v.reshape(1, -1)
    lay = lambda v: v.reshape(_NL - 1, 1, -1)

    eye_h = jnp.eye(_H, dtype=f32)
    Al = (al_gat[:, :, :, None] * eye_h[None, :, None, :]).reshape(
        _NL - 1, _HID, _H)
    Ar = (ar_gat[:, :, :, None] * eye_h[None, :, None, :]).reshape(
        _NL - 1, _HID, _H)
    W_ext = jnp.concatenate(
        [W_gat,
         jnp.einsum('lij,ljk->lik', W_gat, Al),
         jnp.einsum('lij,ljk->lik', W_gat, Ar)], axis=2)
    W_last_ext = jnp.concatenate(
        [W_last, W_last @ al_last.T, W_last @ ar_last.T], axis=1)

    full = lambda shp: pl.BlockSpec(shp, lambda i, g: tuple(0 for _ in shp))
    grp = lambda shp: pl.BlockSpec(
        shp, lambda i, g: (g,) + tuple(0 for _ in shp[1:]))
    per_layer = lambda shp: pl.BlockSpec(
        shp, lambda i, g: (jnp.minimum(i, _NL - 2),) + tuple(0 for _ in shp[1:]))
    prev_layer = lambda shp: pl.BlockSpec(
        shp, lambda i, g: (jnp.clip(i - 1, 0, _NL - 2),)
        + tuple(0 for _ in shp[1:]))

    out = pl.pallas_call(
        _gat_body,
        grid=(_NL + 1, _G),
        in_specs=[
            grp((_S, _N, 2)),
            grp((_S, _N, _NT)),
            grp((_S, 1, _N)),
            full((_NT, _HID)),
            full((1, _HID)),
            per_layer((1, _HID, _HID + 2 * _H)),
            per_layer((1, 1, _HID)),
            prev_layer((1, 1, _HID)),
            prev_layer((1, 1, _HID)),
            full((_HID, _D + 2)),
            full((1, _D)),
            full((1, _D)),
            full((1, _D)),
            full((_D, 35)),
            full((1, 35)),
            full((35, 17)),
            full((1, 17)),
            full((17, 5)),
            full((1, 5)),
        ],
        out_specs=pl.BlockSpec((_B, 5), lambda i, g: (0, 0)),
        out_shape=jax.ShapeDtypeStruct((_B, 5), f32),
        scratch_shapes=[
            pltpu.VMEM((_NTOT, _HID), f32),
            pltpu.VMEM((_NTOT, _HID), f32),
            pltpu.VMEM((_B, _N, _N), jnp.bfloat16),
            pltpu.VMEM((1, _HID), f32),
            pltpu.VMEM((1, _HID), f32),
            pltpu.VMEM((1, _HID), f32),
            pltpu.VMEM((1, _HID), f32),
        ],
        interpret=_INTERPRET,
    )(points.astype(f32), features.astype(f32),
      mask.astype(f32).reshape(_B, 1, _N),
      W_emb, row(b_emb), W_ext, lay(b_gat),
      lay(g_bn), lay(be_bn), W_last_ext, row(b_last),
      row(g_bnl), row(be_bnl), W1, row(b1), W2, row(b2), W3, row(b3))
    return out


# speedup vs baseline: 364.3995x; 1.2863x over previous
import jax
import jax.numpy as jnp
from jax.experimental import pallas as pl
from jax.experimental.pallas import tpu as pltpu

_B, _N, _K = 64, 128, 16
_H, _D = 4, 70
_HID = _H * _D
_NT = 34
_NL = 9
_S = 16
_G = _B // _S
_ROWS = _S * _N
_NTOT = _B * _N

_INTERPRET = False


def _elu(x):
    return jnp.where(x > 0, x, jnp.exp(jnp.minimum(x, 0.0)) - 1.0)


def _leaky(x):
    return jnp.where(x >= 0, x, 0.2 * x)


def _eye_n():
    r = jax.lax.broadcasted_iota(jnp.int32, (_N, _N), 0)
    c = jax.lax.broadcasted_iota(jnp.int32, (_N, _N), 1)
    return jnp.where(r == c, 1.0, 0.0).astype(jnp.float32)


def _transpose_cols(cols, ident):
    return jax.lax.dot_general(cols, ident, (((0,), (0,)), ((), ())),
                               preferred_element_type=jnp.float32,
                               precision=jax.lax.Precision.HIGHEST)


def _gat_body(pts_ref, feat_ref, mask_ref, wemb_ref, bemb_ref, wgat_ref,
              bgat_ref, gbn_ref, bebn_ref, wlast_ref,
              blast_ref, gbnl_ref, bebnl_ref,
              w1_ref, b1_ref, w2_ref, b2_ref, w3_ref, b3_ref,
              y_ref, H_s, P_s, A_s, accs_s, accq_s, mean_s, inv_s):
    i = pl.program_id(0)
    g = pl.program_id(1)
    row0 = g * _ROWS

    @pl.when(jnp.logical_and(i == 0, g == 0))
    def _init_acc():
        accs_s[...] = jnp.zeros_like(accs_s)
        accq_s[...] = jnp.zeros_like(accq_s)

    @pl.when(jnp.logical_and(i > 0, g == 0))
    def _finalize_stats():
        m = accs_s[...] * (1.0 / _NTOT)
        q = accq_s[...] * (1.0 / _NTOT)
        mean_s[...] = m
        inv_s[...] = 1.0 / jnp.sqrt(q - m * m + 1e-5)
        accs_s[...] = jnp.zeros_like(accs_s)
        accq_s[...] = jnp.zeros_like(accq_s)

    @pl.when(i == 0)
    def _knn_and_embed():
        pts = pts_ref[...]
        ident = _eye_n()
        jf = jax.lax.broadcasted_iota(jnp.int32, (_S, _N, _N), 2).astype(
            jnp.float32)
        eye_big = jnp.where(
            jax.lax.broadcasted_iota(jnp.int32, (_N, _N), 0)
            == jax.lax.broadcasted_iota(jnp.int32, (_N, _N), 1), 1e9, 0.0)

        def _extract(_, carry):
            adj_c, cur_c = carry
            mval = jnp.min(cur_c, axis=2, keepdims=True)
            cand = jnp.where(cur_c == mval, jf, 1e9)
            jstar = jnp.min(cand, axis=2, keepdims=True)
            pick = jf == jstar
            return (jnp.where(pick, 1.0, adj_c),
                    jnp.where(pick, 3e9, cur_c))

        ds = []
        for b in range(_S):
            pc = pts[b]
            pr = _transpose_cols(pc, ident)
            dx = pc[:, 0:1] - pr[0:1, :]
            dy = pc[:, 1:2] - pr[1:2, :]
            d = dx * dx + dy * dy + eye_big
            ds.append(d.reshape(1, _N, _N))
        d3 = jnp.concatenate(ds, axis=0)
        adj3, _ = jax.lax.fori_loop(
            0, _K, _extract, (jnp.zeros((_S, _N, _N), jnp.float32), d3))
        A_s[pl.ds(g * _S, _S), :, :] = adj3.astype(jnp.bfloat16)
        feat = feat_ref[...].reshape(_ROWS, _NT)
        h0 = jnp.dot(feat, wemb_ref[...],
                     preferred_element_type=jnp.float32) + bemb_ref[...]
        H_s[pl.ds(row0, _ROWS), :] = h0

    @pl.when(jnp.logical_and(i >= 1, i <= _NL - 1))
    def _apply_bn_residual():
        p = P_s[pl.ds(row0, _ROWS), :]
        x = gbn_ref[0] * (p - mean_s[...]) * inv_s[...] + bebn_ref[0]
        H_s[pl.ds(row0, _ROWS), :] = H_s[pl.ds(row0, _ROWS), :] + _elu(x)

    @pl.when(i <= _NL - 2)
    def _conv_multihead():
        h = H_s[pl.ds(row0, _ROWS), :]
        zext = jnp.dot(h, wgat_ref[0], preferred_element_type=jnp.float32)
        z3e = zext.reshape(_S, _N, _HID + 2 * _H)
        ident = _eye_n()
        for b in range(_S):
            zs = z3e[b]
            adj = A_s[pl.ds(g * _S + b, 1), :, :].reshape(_N, _N)
            el_rows = _transpose_cols(zs[:, _HID:_HID + _H], ident)
            cols = []
            for hh in range(_H):
                e = _leaky(el_rows[hh:hh + 1, :]
                           + zs[:, _HID + _H + hh:_HID + _H + hh + 1])
                em = jnp.where(adj > 0, e, -1e9)
                emax = jnp.max(em, axis=1, keepdims=True)
                ee = jnp.exp(em - emax)
                den = jnp.sum(ee, axis=1, keepdims=True)
                alpha = ee / (den + 1e-9)
                cols.append(jnp.dot(alpha, zs[:, hh * _D:(hh + 1) * _D],
                                    preferred_element_type=jnp.float32))
            rowblk = jnp.concatenate(cols, axis=1) + bgat_ref[0]
            P_s[pl.ds(row0 + b * _N, _N), :] = rowblk
            accs_s[...] += jnp.sum(rowblk, axis=0, keepdims=True)
            accq_s[...] += jnp.sum(rowblk * rowblk, axis=0, keepdims=True)

    @pl.when(i == _NL - 1)
    def _conv_last():
        h = H_s[pl.ds(row0, _ROWS), :]
        zext = jnp.dot(h, wlast_ref[...], preferred_element_type=jnp.float32)
        z3e = zext.reshape(_S, _N, _D + 2)
        ident = _eye_n()
        zero_pad = jnp.zeros((_N, _HID - _D), jnp.float32)
        for b in range(_S):
            zs = z3e[b]
            adj = A_s[pl.ds(g * _S + b, 1), :, :].reshape(_N, _N)
            el_row = _transpose_cols(zs[:, _D:_D + 1], ident)
            e = _leaky(el_row + zs[:, _D + 1:_D + 2])
            em = jnp.where(adj > 0, e, -1e9)
            emax = jnp.max(em, axis=1, keepdims=True)
            ee = jnp.exp(em - emax)
            den = jnp.sum(ee, axis=1, keepdims=True)
            alpha = ee / (den + 1e-9)
            ob = jnp.dot(alpha, zs[:, 0:_D],
                         preferred_element_type=jnp.float32) + blast_ref[...]
            rowblk = jnp.concatenate([ob, zero_pad], axis=1)
            P_s[pl.ds(row0 + b * _N, _N), :] = rowblk
            accs_s[...] += jnp.sum(rowblk, axis=0, keepdims=True)
            accq_s[...] += jnp.sum(rowblk * rowblk, axis=0, keepdims=True)

    @pl.when(i == _NL)
    def _readout():
        p = P_s[pl.ds(row0, _ROWS), :][:, 0:_D]
        x = (gbnl_ref[...] * (p - mean_s[0:1, 0:_D]) * inv_s[0:1, 0:_D]
             + bebnl_ref[...])
        hn3 = _elu(x).reshape(_S, _N, _D)
        mk = mask_ref[...]
        rows = []
        for b in range(_S):
            mk_s = mk[b]
            tot = jnp.dot(mk_s, hn3[b], preferred_element_type=jnp.float32)
            cnt = jnp.sum(mk_s, axis=1, keepdims=True) + 1e-9
            rows.append(tot / cnt)
        hg = jnp.concatenate(rows, axis=0)
        y1 = jnp.maximum(jnp.dot(hg, w1_ref[...],
                                 preferred_element_type=jnp.float32)
                         + b1_ref[...], 0.0)
        y2 = jnp.maximum(jnp.dot(y1, w2_ref[...],
                                 preferred_element_type=jnp.float32)
                         + b2_ref[...], 0.0)
        y_ref[pl.ds(g * _S, _S), :] = jnp.dot(
            y2, w3_ref[...], preferred_element_type=jnp.float32) + b3_ref[...]


def kernel(points, features, lorentz_vectors, mask, W_emb, b_emb, W_gat,
           al_gat, ar_gat, b_gat, g_bn, be_bn, W_last, al_last, ar_last,
           b_last, g_bnl, be_bnl, W1, b1, W2, b2, W3, b3):
    del lorentz_vectors
    f32 = jnp.float32
    row = lambda v: v.reshape(1, -1)
    lay = lambda v: v.reshape(_NL - 1, 1, -1)

    eye_h = jnp.eye(_H, dtype=f32)
    Al = (al_gat[:, :, :, None] * eye_h[None, :, None, :]).reshape(
        _NL - 1, _HID, _H)
    Ar = (ar_gat[:, :, :, None] * eye_h[None, :, None, :]).reshape(
        _NL - 1, _HID, _H)
    W_ext = jnp.concatenate(
        [W_gat,
         jnp.einsum('lij,ljk->lik', W_gat, Al),
         jnp.einsum('lij,ljk->lik', W_gat, Ar)], axis=2)
    W_last_ext = jnp.concatenate(
        [W_last, W_last @ al_last.T, W_last @ ar_last.T], axis=1)

    full = lambda shp: pl.BlockSpec(shp, lambda i, g: tuple(0 for _ in shp))
    grp = lambda shp: pl.BlockSpec(
        shp, lambda i, g: (g,) + tuple(0 for _ in shp[1:]))
    per_layer = lambda shp: pl.BlockSpec(
        shp, lambda i, g: (jnp.minimum(i, _NL - 2),) + tuple(0 for _ in shp[1:]))
    prev_layer = lambda shp: pl.BlockSpec(
        shp, lambda i, g: (jnp.clip(i - 1, 0, _NL - 2),)
        + tuple(0 for _ in shp[1:]))

    out = pl.pallas_call(
        _gat_body,
        grid=(_NL + 1, _G),
        in_specs=[
            grp((_S, _N, 2)),
            grp((_S, _N, _NT)),
            grp((_S, 1, _N)),
            full((_NT, _HID)),
            full((1, _HID)),
            per_layer((1, _HID, _HID + 2 * _H)),
            per_layer((1, 1, _HID)),
            prev_layer((1, 1, _HID)),
            prev_layer((1, 1, _HID)),
            full((_HID, _D + 2)),
            full((1, _D)),
            full((1, _D)),
            full((1, _D)),
            full((_D, 35)),
            full((1, 35)),
            full((35, 17)),
            full((1, 17)),
            full((17, 5)),
            full((1, 5)),
        ],
        out_specs=pl.BlockSpec((_B, 5), lambda i, g: (0, 0)),
        out_shape=jax.ShapeDtypeStruct((_B, 5), f32),
        scratch_shapes=[
            pltpu.VMEM((_NTOT, _HID), f32),
            pltpu.VMEM((_NTOT, _HID), f32),
            pltpu.VMEM((_B, _N, _N), jnp.bfloat16),
            pltpu.VMEM((1, _HID), f32),
            pltpu.VMEM((1, _HID), f32),
            pltpu.VMEM((1, _HID), f32),
            pltpu.VMEM((1, _HID), f32),
        ],
        interpret=_INTERPRET,
    )(points.astype(f32), features.astype(f32),
      mask.astype(f32).reshape(_B, 1, _N),
      W_emb, row(b_emb), W_ext, lay(b_gat),
      lay(g_bn), lay(be_bn), W_last_ext, row(b_last),
      row(g_bnl), row(be_bnl), W1, row(b1), W2, row(b2), W3, row(b3))
    return out


# speedup vs baseline: 369.2068x; 1.0132x over previous
import jax
import jax.numpy as jnp
from jax.experimental import pallas as pl
from jax.experimental.pallas import tpu as pltpu

_B, _N, _K = 64, 128, 16
_H, _D = 4, 70
_HID = _H * _D
_NT = 34
_NL = 9
_S = 16
_G = _B // _S
_ROWS = _S * _N
_NTOT = _B * _N

_INTERPRET = False


def _elu(x):
    return jnp.where(x > 0, x, jnp.exp(jnp.minimum(x, 0.0)) - 1.0)


def _leaky(x):
    return jnp.where(x >= 0, x, 0.2 * x)


def _eye_n():
    r = jax.lax.broadcasted_iota(jnp.int32, (_N, _N), 0)
    c = jax.lax.broadcasted_iota(jnp.int32, (_N, _N), 1)
    return jnp.where(r == c, 1.0, 0.0).astype(jnp.float32)


def _transpose_cols(cols, ident):
    return jax.lax.dot_general(cols, ident, (((0,), (0,)), ((), ())),
                               preferred_element_type=jnp.float32,
                               precision=jax.lax.Precision.HIGHEST)


def _gat_body(pts_ref, feat_ref, mask_ref, wemb_ref, bemb_ref, wgat_ref,
              bgat_ref, gbn_ref, bebn_ref, wlast_ref,
              blast_ref, gbnl_ref, bebnl_ref,
              w1_ref, b1_ref, w2_ref, b2_ref, w3_ref, b3_ref,
              y_ref, H_s, P_s, A_s, accs_s, accq_s, mean_s, inv_s):
    i = pl.program_id(0)
    g = pl.program_id(1)
    row0 = g * _ROWS

    @pl.when(jnp.logical_and(i == 0, g == 0))
    def _init_acc():
        accs_s[...] = jnp.zeros_like(accs_s)
        accq_s[...] = jnp.zeros_like(accq_s)

    @pl.when(jnp.logical_and(i > 0, g == 0))
    def _finalize_stats():
        m = accs_s[...] * (1.0 / _NTOT)
        q = accq_s[...] * (1.0 / _NTOT)
        inv = 1.0 / jnp.sqrt(q - m * m + 1e-5)

        @pl.when(i <= _NL - 1)
        def _fold_hidden():
            scale = gbn_ref[0] * inv
            inv_s[...] = scale
            mean_s[...] = bebn_ref[0] - m * scale

        @pl.when(i == _NL)
        def _fold_last():
            scale = gbnl_ref[...] * inv[0:1, 0:_D]
            inv_s[0:1, 0:_D] = scale
            mean_s[0:1, 0:_D] = bebnl_ref[...] - m[0:1, 0:_D] * scale

        accs_s[...] = jnp.zeros_like(accs_s)
        accq_s[...] = jnp.zeros_like(accq_s)

    @pl.when(i == 0)
    def _knn_and_embed():
        pts = pts_ref[...]
        ident = _eye_n()
        jf = jax.lax.broadcasted_iota(jnp.int32, (_S, _N, _N), 2).astype(
            jnp.float32)
        eye_big = jnp.where(
            jax.lax.broadcasted_iota(jnp.int32, (_N, _N), 0)
            == jax.lax.broadcasted_iota(jnp.int32, (_N, _N), 1), 1e9, 0.0)

        def _extract(_, carry):
            adj_c, cur_c = carry
            mval = jnp.min(cur_c, axis=2, keepdims=True)
            cand = jnp.where(cur_c == mval, jf, 1e9)
            jstar = jnp.min(cand, axis=2, keepdims=True)
            pick = jf == jstar
            return (jnp.where(pick, 0.0, adj_c),
                    jnp.where(pick, 3e9, cur_c))

        ds = []
        for b in range(_S):
            pc = pts[b]
            pr = _transpose_cols(pc, ident)
            dx = pc[:, 0:1] - pr[0:1, :]
            dy = pc[:, 1:2] - pr[1:2, :]
            d = dx * dx + dy * dy + eye_big
            ds.append(d.reshape(1, _N, _N))
        d3 = jnp.concatenate(ds, axis=0)
        adj3, _ = jax.lax.fori_loop(
            0, _K, _extract, (jnp.full((_S, _N, _N), -1e9, jnp.float32), d3))
        A_s[pl.ds(g * _S, _S), :, :] = adj3
        feat = feat_ref[...].reshape(_ROWS, _NT)
        h0 = jnp.dot(feat, wemb_ref[...],
                     preferred_element_type=jnp.float32) + bemb_ref[...]
        H_s[pl.ds(row0, _ROWS), :] = h0

    @pl.when(jnp.logical_and(i >= 1, i <= _NL - 1))
    def _apply_bn_residual():
        p = P_s[pl.ds(row0, _ROWS), :]
        x = p * inv_s[...] + mean_s[...]
        H_s[pl.ds(row0, _ROWS), :] = H_s[pl.ds(row0, _ROWS), :] + _elu(x)

    @pl.when(i <= _NL - 2)
    def _conv_multihead():
        h = H_s[pl.ds(row0, _ROWS), :]
        zext = jnp.dot(h, wgat_ref[0], preferred_element_type=jnp.float32)
        z3e = zext.reshape(_S, _N, _HID + 2 * _H)
        ident = _eye_n()
        ssum = jnp.zeros((1, _HID), jnp.float32)
        sqsum = jnp.zeros((1, _HID), jnp.float32)
        for b in range(_S):
            zs = z3e[b]
            adj = A_s[pl.ds(g * _S + b, 1), :, :].reshape(_N, _N)
            el_rows = _transpose_cols(zs[:, _HID:_HID + _H], ident)
            cols = []
            for hh in range(_H):
                e = _leaky(el_rows[hh:hh + 1, :]
                           + zs[:, _HID + _H + hh:_HID + _H + hh + 1])
                em = e + adj
                emax = jnp.max(em, axis=1, keepdims=True)
                ee = jnp.exp(em - emax)
                den = jnp.sum(ee, axis=1, keepdims=True)
                alpha = ee / (den + 1e-9)
                cols.append(jnp.dot(alpha, zs[:, hh * _D:(hh + 1) * _D],
                                    preferred_element_type=jnp.float32))
            rowblk = jnp.concatenate(cols, axis=1) + bgat_ref[0]
            P_s[pl.ds(row0 + b * _N, _N), :] = rowblk
            ssum = ssum + jnp.sum(rowblk, axis=0, keepdims=True)
            sqsum = sqsum + jnp.sum(rowblk * rowblk, axis=0, keepdims=True)
        accs_s[...] += ssum
        accq_s[...] += sqsum

    @pl.when(i == _NL - 1)
    def _conv_last():
        h = H_s[pl.ds(row0, _ROWS), :]
        zext = jnp.dot(h, wlast_ref[...], preferred_element_type=jnp.float32)
        z3e = zext.reshape(_S, _N, _D + 2)
        ident = _eye_n()
        zero_pad = jnp.zeros((_N, _HID - _D), jnp.float32)
        ssum = jnp.zeros((1, _HID), jnp.float32)
        sqsum = jnp.zeros((1, _HID), jnp.float32)
        for b in range(_S):
            zs = z3e[b]
            adj = A_s[pl.ds(g * _S + b, 1), :, :].reshape(_N, _N)
            el_row = _transpose_cols(zs[:, _D:_D + 1], ident)
            e = _leaky(el_row + zs[:, _D + 1:_D + 2])
            em = e + adj
            emax = jnp.max(em, axis=1, keepdims=True)
            ee = jnp.exp(em - emax)
            den = jnp.sum(ee, axis=1, keepdims=True)
            alpha = ee / (den + 1e-9)
            ob = jnp.dot(alpha, zs[:, 0:_D],
                         preferred_element_type=jnp.float32) + blast_ref[...]
            rowblk = jnp.concatenate([ob, zero_pad], axis=1)
            P_s[pl.ds(row0 + b * _N, _N), :] = rowblk
            ssum = ssum + jnp.sum(rowblk, axis=0, keepdims=True)
            sqsum = sqsum + jnp.sum(rowblk * rowblk, axis=0, keepdims=True)
        accs_s[...] += ssum
        accq_s[...] += sqsum

    @pl.when(i == _NL)
    def _readout():
        p = P_s[pl.ds(row0, _ROWS), :][:, 0:_D]
        x = p * inv_s[0:1, 0:_D] + mean_s[0:1, 0:_D]
        hn3 = _elu(x).reshape(_S, _N, _D)
        mk = mask_ref[...]
        rows = []
        for b in range(_S):
            mk_s = mk[b]
            tot = jnp.dot(mk_s, hn3[b], preferred_element_type=jnp.float32)
            cnt = jnp.sum(mk_s, axis=1, keepdims=True) + 1e-9
            rows.append(tot / cnt)
        hg = jnp.concatenate(rows, axis=0)
        y1 = jnp.maximum(jnp.dot(hg, w1_ref[...],
                                 preferred_element_type=jnp.float32)
                         + b1_ref[...], 0.0)
        y2 = jnp.maximum(jnp.dot(y1, w2_ref[...],
                                 preferred_element_type=jnp.float32)
                         + b2_ref[...], 0.0)
        y_ref[pl.ds(g * _S, _S), :] = jnp.dot(
            y2, w3_ref[...], preferred_element_type=jnp.float32) + b3_ref[...]


def kernel(points, features, lorentz_vectors, mask, W_emb, b_emb, W_gat,
           al_gat, ar_gat, b_gat, g_bn, be_bn, W_last, al_last, ar_last,
           b_last, g_bnl, be_bnl, W1, b1, W2, b2, W3, b3):
    del lorentz_vectors
    f32 = jnp.float32
    row = lambda v: v.reshape(1, -1)
    lay = lambda v: v.reshape(_NL - 1, 1, -1)

    eye_h = jnp.eye(_H, dtype=f32)
    Al = (al_gat[:, :, :, None] * eye_h[None, :, None, :]).reshape(
        _NL - 1, _HID, _H)
    Ar = (ar_gat[:, :, :, None] * eye_h[None, :, None, :]).reshape(
        _NL - 1, _HID, _H)
    W_ext = jnp.concatenate(
        [W_gat,
         jnp.einsum('lij,ljk->lik', W_gat, Al),
         jnp.einsum('lij,ljk->lik', W_gat, Ar)], axis=2)
    W_last_ext = jnp.concatenate(
        [W_last, W_last @ al_last.T, W_last @ ar_last.T], axis=1)

    full = lambda shp: pl.BlockSpec(shp, lambda i, g: tuple(0 for _ in shp))
    grp = lambda shp: pl.BlockSpec(
        shp, lambda i, g: (g,) + tuple(0 for _ in shp[1:]))
    per_layer = lambda shp: pl.BlockSpec(
        shp, lambda i, g: (jnp.minimum(i, _NL - 2),) + tuple(0 for _ in shp[1:]))
    prev_layer = lambda shp: pl.BlockSpec(
        shp, lambda i, g: (jnp.clip(i - 1, 0, _NL - 2),)
        + tuple(0 for _ in shp[1:]))

    out = pl.pallas_call(
        _gat_body,
        grid=(_NL + 1, _G),
        in_specs=[
            grp((_S, _N, 2)),
            grp((_S, _N, _NT)),
            grp((_S, 1, _N)),
            full((_NT, _HID)),
            full((1, _HID)),
            per_layer((1, _HID, _HID + 2 * _H)),
            per_layer((1, 1, _HID)),
            prev_layer((1, 1, _HID)),
            prev_layer((1, 1, _HID)),
            full((_HID, _D + 2)),
            full((1, _D)),
            full((1, _D)),
            full((1, _D)),
            full((_D, 35)),
            full((1, 35)),
            full((35, 17)),
            full((1, 17)),
            full((17, 5)),
            full((1, 5)),
        ],
        out_specs=pl.BlockSpec((_B, 5), lambda i, g: (0, 0)),
        out_shape=jax.ShapeDtypeStruct((_B, 5), f32),
        scratch_shapes=[
            pltpu.VMEM((_NTOT, _HID), f32),
            pltpu.VMEM((_NTOT, _HID), f32),
            pltpu.VMEM((_B, _N, _N), f32),
            pltpu.VMEM((1, _HID), f32),
            pltpu.VMEM((1, _HID), f32),
            pltpu.VMEM((1, _HID), f32),
            pltpu.VMEM((1, _HID), f32),
        ],
        interpret=_INTERPRET,
    )(points.astype(f32), features.astype(f32),
      mask.astype(f32).reshape(_B, 1, _N),
      W_emb, row(b_emb), W_ext, lay(b_gat),
      lay(g_bn), lay(be_bn), W_last_ext, row(b_last),
      row(g_bnl), row(be_bnl), W1, row(b1), W2, row(b2), W3, row(b3))
    return out
